# baseline (speedup 1.0000x reference)
import functools

import jax
import jax.numpy as jnp
from jax import lax
from jax.experimental import pallas as pl
from jax.experimental.pallas import tpu as pltpu


def _matmul_stats_kernel(x_ref, w_ref, y_ref, sum_ref, ssq_ref):
    y = jnp.dot(x_ref[...].astype(jnp.bfloat16), w_ref[...],
                preferred_element_type=jnp.float32)
    sum_ref[...] = jnp.sum(y, axis=0, keepdims=True)[None]
    ssq_ref[...] = jnp.sum(y * y, axis=0, keepdims=True)[None]
    y_ref[...] = y.astype(jnp.bfloat16)


def _apply_kernel(y_ref, sum_ref, ssq_ref, gamma_ref, beta_ref, o_ref,
                  scale_ref, bias_ref, *,
                  count, num_group, cg, eps, negative_slope):
    @pl.when(pl.program_id(1) == 0)
    def _():
        dout = num_group * cg
        chan = lax.broadcasted_iota(jnp.int32, (dout, num_group), 0)
        grp = lax.broadcasted_iota(jnp.int32, (dout, num_group), 1)
        g_onehot = (chan // cg == grp).astype(jnp.float32)
        sum_c = jnp.sum(sum_ref[...], axis=0)
        ssq_c = jnp.sum(ssq_ref[...], axis=0)
        g_sum = jnp.dot(sum_c, g_onehot, preferred_element_type=jnp.float32)
        g_ssq = jnp.dot(ssq_c, g_onehot, preferred_element_type=jnp.float32)
        mean_g = g_sum / count
        var_g = jnp.maximum(g_ssq / count - mean_g * mean_g, 0.0)
        inv_g = lax.rsqrt(var_g + eps)
        inv_c = jnp.dot(inv_g, g_onehot.T, preferred_element_type=jnp.float32)
        mean_c = jnp.dot(mean_g, g_onehot.T, preferred_element_type=jnp.float32)
        scale_ref[...] = gamma_ref[...] * inv_c
        bias_ref[...] = beta_ref[...] - mean_c * scale_ref[...]

    z = y_ref[...].astype(jnp.float32) * scale_ref[...] + bias_ref[...]
    z = jnp.maximum(z, negative_slope * z)
    o_ref[...] = z.astype(o_ref.dtype)


def _pick_tile(n, cap):
    for t in range(cap, 7, -8):
        if n % t == 0 and (n // t) % 2 == 0:
            return t
    return None


def kernel(x, w, gamma, beta):
    num_group = 32
    eps = 1e-5
    negative_slope = 0.1

    n, din = x.shape
    dout = w.shape[0]
    cg = dout // num_group

    tile_s = _pick_tile(n, 5000)
    if tile_s is None:
        tile_s = 2048
        num_tiles = pl.cdiv(n, tile_s)
        num_tiles += num_tiles % 2
        n_pad = num_tiles * tile_s
        x_pad = jnp.pad(x, ((0, n_pad - n), (0, 0)))
    else:
        num_tiles = n // tile_s
        n_pad = n
        x_pad = x

    w_t = jnp.transpose(w).astype(jnp.bfloat16)

    y_bf16, sum_pt, ssq_pt = pl.pallas_call(
        _matmul_stats_kernel,
        out_shape=(jax.ShapeDtypeStruct((n_pad, dout), jnp.bfloat16),
                   jax.ShapeDtypeStruct((num_tiles, 1, dout), jnp.float32),
                   jax.ShapeDtypeStruct((num_tiles, 1, dout), jnp.float32)),
        grid=(num_tiles,),
        in_specs=[
            pl.BlockSpec((tile_s, din), lambda i: (i, 0)),
            pl.BlockSpec((din, dout), lambda i: (0, 0)),
        ],
        out_specs=(
            pl.BlockSpec((tile_s, dout), lambda i: (i, 0)),
            pl.BlockSpec((1, 1, dout), lambda i: (i, 0, 0)),
            pl.BlockSpec((1, 1, dout), lambda i: (i, 0, 0)),
        ),
        compiler_params=pltpu.CompilerParams(
            dimension_semantics=("parallel",)),
    )(x_pad, w_t)

    tile_a = _pick_tile(n_pad, 5000) or tile_s
    num_tiles_a = n_pad // tile_a
    half_a = num_tiles_a // 2
    apply_fn = functools.partial(
        _apply_kernel, count=float(n) * cg, num_group=num_group, cg=cg,
        eps=eps, negative_slope=negative_slope)
    out_pad = pl.pallas_call(
        apply_fn,
        out_shape=jax.ShapeDtypeStruct((n_pad, dout), x.dtype),
        grid=(2, half_a),
        in_specs=[
            pl.BlockSpec((tile_a, dout), lambda i, j: (i * half_a + j, 0)),
            pl.BlockSpec((num_tiles, 1, dout), lambda i, j: (0, 0, 0)),
            pl.BlockSpec((num_tiles, 1, dout), lambda i, j: (0, 0, 0)),
            pl.BlockSpec((1, dout), lambda i, j: (0, 0)),
            pl.BlockSpec((1, dout), lambda i, j: (0, 0)),
        ],
        out_specs=pl.BlockSpec((tile_a, dout), lambda i, j: (i * half_a + j, 0)),
        scratch_shapes=[
            pltpu.VMEM((1, dout), jnp.float32),
            pltpu.VMEM((1, dout), jnp.float32),
        ],
        compiler_params=pltpu.CompilerParams(
            dimension_semantics=("parallel", "arbitrary")),
    )(y_bf16, sum_pt, ssq_pt, gamma.reshape(1, dout), beta.reshape(1, dout))

    out = out_pad if n_pad == n else out_pad[:n]
    return jnp.squeeze(out)


# speedup vs baseline: 1.0201x; 1.0201x over previous
import functools

import jax
import jax.numpy as jnp
from jax import lax
from jax.experimental import pallas as pl
from jax.experimental.pallas import tpu as pltpu


def _matmul_stats_kernel(x_ref, w_ref, y_ref, sum_ref, ssq_ref):
    y = lax.dot_general(
        x_ref[...].astype(jnp.bfloat16), w_ref[...].astype(jnp.bfloat16),
        dimension_numbers=(((1,), (1,)), ((), ())),
        preferred_element_type=jnp.float32)
    sum_ref[...] = jnp.sum(y, axis=0, keepdims=True)[None]
    ssq_ref[...] = jnp.sum(y * y, axis=0, keepdims=True)[None]
    y_ref[...] = y.astype(jnp.bfloat16)


def _apply_kernel(y_ref, sum_ref, ssq_ref, gamma_ref, beta_ref, o_ref,
                  scale_ref, bias_ref, *,
                  count, num_group, cg, eps, negative_slope):
    @pl.when(pl.program_id(1) == 0)
    def _():
        dout = num_group * cg
        chan = lax.broadcasted_iota(jnp.int32, (dout, num_group), 0)
        grp = lax.broadcasted_iota(jnp.int32, (dout, num_group), 1)
        g_onehot = (chan // cg == grp).astype(jnp.float32)
        sum_c = jnp.sum(sum_ref[...], axis=0)
        ssq_c = jnp.sum(ssq_ref[...], axis=0)
        g_sum = jnp.dot(sum_c, g_onehot, preferred_element_type=jnp.float32)
        g_ssq = jnp.dot(ssq_c, g_onehot, preferred_element_type=jnp.float32)
        mean_g = g_sum / count
        var_g = jnp.maximum(g_ssq / count - mean_g * mean_g, 0.0)
        inv_g = lax.rsqrt(var_g + eps)
        inv_c = jnp.dot(inv_g, g_onehot.T, preferred_element_type=jnp.float32)
        mean_c = jnp.dot(mean_g, g_onehot.T, preferred_element_type=jnp.float32)
        scale_ref[...] = gamma_ref[...] * inv_c
        bias_ref[...] = beta_ref[...] - mean_c * scale_ref[...]

    z = y_ref[...].astype(jnp.float32) * scale_ref[...] + bias_ref[...]
    z = jnp.maximum(z, negative_slope * z)
    o_ref[...] = z.astype(o_ref.dtype)


def _pick_tile(n, cap):
    for t in range(cap, 7, -8):
        if n % t == 0 and (n // t) % 2 == 0:
            return t
    return None


def kernel(x, w, gamma, beta):
    num_group = 32
    eps = 1e-5
    negative_slope = 0.1

    n, din = x.shape
    dout = w.shape[0]
    cg = dout // num_group

    tile_s = _pick_tile(n, 5000)
    if tile_s is None:
        tile_s = 2048
        num_tiles = pl.cdiv(n, tile_s)
        num_tiles += num_tiles % 2
        n_pad = num_tiles * tile_s
        x_pad = jnp.pad(x, ((0, n_pad - n), (0, 0)))
    else:
        num_tiles = n // tile_s
        n_pad = n
        x_pad = x

    y_bf16, sum_pt, ssq_pt = pl.pallas_call(
        _matmul_stats_kernel,
        out_shape=(jax.ShapeDtypeStruct((n_pad, dout), jnp.bfloat16),
                   jax.ShapeDtypeStruct((num_tiles, 1, dout), jnp.float32),
                   jax.ShapeDtypeStruct((num_tiles, 1, dout), jnp.float32)),
        grid=(num_tiles,),
        in_specs=[
            pl.BlockSpec((tile_s, din), lambda i: (i, 0)),
            pl.BlockSpec((dout, din), lambda i: (0, 0)),
        ],
        out_specs=(
            pl.BlockSpec((tile_s, dout), lambda i: (i, 0)),
            pl.BlockSpec((1, 1, dout), lambda i: (i, 0, 0)),
            pl.BlockSpec((1, 1, dout), lambda i: (i, 0, 0)),
        ),
        compiler_params=pltpu.CompilerParams(
            dimension_semantics=("parallel",)),
    )(x_pad, w)

    tile_a = _pick_tile(n_pad, 5000) or tile_s
    num_tiles_a = n_pad // tile_a
    half_a = num_tiles_a // 2
    apply_fn = functools.partial(
        _apply_kernel, count=float(n) * cg, num_group=num_group, cg=cg,
        eps=eps, negative_slope=negative_slope)
    out_pad = pl.pallas_call(
        apply_fn,
        out_shape=jax.ShapeDtypeStruct((n_pad, dout), x.dtype),
        grid=(2, half_a),
        in_specs=[
            pl.BlockSpec((tile_a, dout), lambda i, j: (i * half_a + j, 0)),
            pl.BlockSpec((num_tiles, 1, dout), lambda i, j: (0, 0, 0)),
            pl.BlockSpec((num_tiles, 1, dout), lambda i, j: (0, 0, 0)),
            pl.BlockSpec((1, dout), lambda i, j: (0, 0)),
            pl.BlockSpec((1, dout), lambda i, j: (0, 0)),
        ],
        out_specs=pl.BlockSpec((tile_a, dout), lambda i, j: (i * half_a + j, 0)),
        scratch_shapes=[
            pltpu.VMEM((1, dout), jnp.float32),
            pltpu.VMEM((1, dout), jnp.float32),
        ],
        compiler_params=pltpu.CompilerParams(
            dimension_semantics=("parallel", "arbitrary")),
    )(y_bf16, sum_pt, ssq_pt, gamma.reshape(1, dout), beta.reshape(1, dout))

    out = out_pad if n_pad == n else out_pad[:n]
    return jnp.squeeze(out)


# speedup vs baseline: 1.0236x; 1.0034x over previous
import functools

import jax
import jax.numpy as jnp
from jax import lax
from jax.experimental import pallas as pl
from jax.experimental.pallas import tpu as pltpu


def _matmul_stats_kernel(x_ref, w_ref, y_ref, sum_ref, ssq_ref):
    @pl.when(pl.program_id(1) == 0)
    def _():
        sum_ref[...] = jnp.zeros_like(sum_ref)
        ssq_ref[...] = jnp.zeros_like(ssq_ref)

    y = lax.dot_general(
        x_ref[...].astype(jnp.bfloat16), w_ref[...].astype(jnp.bfloat16),
        dimension_numbers=(((1,), (1,)), ((), ())),
        preferred_element_type=jnp.float32)
    sum_ref[...] += jnp.sum(y, axis=0, keepdims=True)[None]
    ssq_ref[...] += jnp.sum(y * y, axis=0, keepdims=True)[None]
    y_ref[...] = y.astype(jnp.bfloat16)


def _apply_kernel(y_ref, sum_ref, ssq_ref, gamma_ref, beta_ref, o_ref,
                  scale_ref, bias_ref, *,
                  count, num_group, cg, eps, negative_slope):
    @pl.when(pl.program_id(1) == 0)
    def _():
        dout = num_group * cg
        chan = lax.broadcasted_iota(jnp.int32, (dout, num_group), 0)
        grp = lax.broadcasted_iota(jnp.int32, (dout, num_group), 1)
        g_onehot = (chan // cg == grp).astype(jnp.float32)
        sum_c = jnp.sum(sum_ref[...], axis=0)
        ssq_c = jnp.sum(ssq_ref[...], axis=0)
        g_sum = jnp.dot(sum_c, g_onehot, preferred_element_type=jnp.float32)
        g_ssq = jnp.dot(ssq_c, g_onehot, preferred_element_type=jnp.float32)
        mean_g = g_sum / count
        var_g = jnp.maximum(g_ssq / count - mean_g * mean_g, 0.0)
        inv_g = lax.rsqrt(var_g + eps)
        inv_c = jnp.dot(inv_g, g_onehot.T, preferred_element_type=jnp.float32)
        mean_c = jnp.dot(mean_g, g_onehot.T, preferred_element_type=jnp.float32)
        scale_ref[...] = gamma_ref[...] * inv_c
        bias_ref[...] = beta_ref[...] - mean_c * scale_ref[...]

    z = y_ref[...].astype(jnp.float32) * scale_ref[...] + bias_ref[...]
    z = jnp.maximum(z, negative_slope * z)
    o_ref[...] = z.astype(o_ref.dtype)


def _pick_tile(n, cap):
    for t in range(cap, 7, -8):
        if n % t == 0 and (n // t) % 2 == 0:
            return t
    return None


def kernel(x, w, gamma, beta):
    num_group = 32
    eps = 1e-5
    negative_slope = 0.1

    n, din = x.shape
    dout = w.shape[0]
    cg = dout // num_group

    tile_s = _pick_tile(n, 5000)
    if tile_s is None:
        tile_s = 2048
        num_tiles = pl.cdiv(n, tile_s)
        num_tiles += num_tiles % 2
        n_pad = num_tiles * tile_s
        x_pad = jnp.pad(x, ((0, n_pad - n), (0, 0)))
    else:
        num_tiles = n // tile_s
        n_pad = n
        x_pad = x

    half_s = num_tiles // 2
    y_bf16, sum_pt, ssq_pt = pl.pallas_call(
        _matmul_stats_kernel,
        out_shape=(jax.ShapeDtypeStruct((n_pad, dout), jnp.bfloat16),
                   jax.ShapeDtypeStruct((2, 1, dout), jnp.float32),
                   jax.ShapeDtypeStruct((2, 1, dout), jnp.float32)),
        grid=(2, half_s),
        in_specs=[
            pl.BlockSpec((tile_s, din), lambda i, j: (i * half_s + j, 0)),
            pl.BlockSpec((dout, din), lambda i, j: (0, 0)),
        ],
        out_specs=(
            pl.BlockSpec((tile_s, dout), lambda i, j: (i * half_s + j, 0)),
            pl.BlockSpec((1, 1, dout), lambda i, j: (i, 0, 0)),
            pl.BlockSpec((1, 1, dout), lambda i, j: (i, 0, 0)),
        ),
        compiler_params=pltpu.CompilerParams(
            dimension_semantics=("parallel", "arbitrary")),
    )(x_pad, w)

    tile_a = _pick_tile(n_pad, 5000) or tile_s
    num_tiles_a = n_pad // tile_a
    half_a = num_tiles_a // 2
    apply_fn = functools.partial(
        _apply_kernel, count=float(n) * cg, num_group=num_group, cg=cg,
        eps=eps, negative_slope=negative_slope)
    out_pad = pl.pallas_call(
        apply_fn,
        out_shape=jax.ShapeDtypeStruct((n_pad, dout), x.dtype),
        grid=(2, half_a),
        in_specs=[
            pl.BlockSpec((tile_a, dout), lambda i, j: (i * half_a + j, 0)),
            pl.BlockSpec((2, 1, dout), lambda i, j: (0, 0, 0)),
            pl.BlockSpec((2, 1, dout), lambda i, j: (0, 0, 0)),
            pl.BlockSpec((1, dout), lambda i, j: (0, 0)),
            pl.BlockSpec((1, dout), lambda i, j: (0, 0)),
        ],
        out_specs=pl.BlockSpec((tile_a, dout), lambda i, j: (i * half_a + j, 0)),
        scratch_shapes=[
            pltpu.VMEM((1, dout), jnp.float32),
            pltpu.VMEM((1, dout), jnp.float32),
        ],
        compiler_params=pltpu.CompilerParams(
            dimension_semantics=("parallel", "arbitrary")),
    )(y_bf16, sum_pt, ssq_pt, gamma.reshape(1, dout), beta.reshape(1, dout))

    out = out_pad if n_pad == n else out_pad[:n]
    return jnp.squeeze(out)
